# Initial kernel scaffold; baseline (speedup 1.0000x reference)
import jax
import jax.numpy as jnp
from jax.experimental import pallas as pl


def kernel(input_features, input_features_global, batch_index):
    raise NotImplementedError("write your pallas kernel here")


# speedup vs baseline: 1.2515x; 1.2515x over previous
import functools

import jax
import jax.numpy as jnp
from jax import lax
from jax.experimental import pallas as pl
from jax.experimental.pallas import tpu as pltpu
from jax.experimental.pallas import tpu_sc as plsc

N = 1000000
C = 64
B = 16

R = 400
CH = N // R
NW = 32
RC = R * C


def _body(feat_hbm, glob_hbm, idx_hbm, out_hbm,
          table_v, bufs0, bufs1, bufs2, idxs0, idxs1, idxs2,
          in_sems, idx_sems, out_sems):
    bufs = (bufs0, bufs1, bufs2)
    idxs = (idxs0, idxs1, idxs2)
    wid = lax.axis_index("s") * 2 + lax.axis_index("c")
    nk = (CH - wid + NW - 1) // NW

    pltpu.sync_copy(glob_hbm, table_v)

    def start_in(t, s):
        g = wid + NW * t
        pltpu.async_copy(feat_hbm.at[pl.ds(g * RC, RC)], bufs[s],
                         in_sems.at[s])
        pltpu.async_copy(idx_hbm.at[pl.ds(g * R, R)], idxs[s],
                         idx_sems.at[s])

    def wait_in(s):
        pltpu.make_async_copy(feat_hbm.at[pl.ds(0, RC)], bufs[s],
                              in_sems.at[s]).wait()
        pltpu.make_async_copy(idx_hbm.at[pl.ds(0, R)], idxs[s],
                              idx_sems.at[s]).wait()

    def start_out(t, s):
        g = wid + NW * t
        pltpu.async_copy(bufs[s], out_hbm.at[pl.ds(g * RC, RC)],
                         out_sems.at[s])

    def wait_out(s):
        pltpu.make_async_copy(bufs[s], out_hbm.at[pl.ds(0, RC)],
                              out_sems.at[s]).wait()

    def compute(s):
        buf = bufs[s]
        idx = idxs[s]

        def group(g, _):
            bvec = idx[pl.ds(g * 16, 16)]
            base = g * (16 * C)
            for r in range(16):
                o = bvec[r] * C
                p = base + r * C
                for k in range(4):
                    buf[pl.ds(p + k * 16, 16)] += table_v[pl.ds(o + k * 16, 16)]
            return 0

        lax.fori_loop(0, R // 16, group, 0)

    def step(t, s, first):
        @pl.when(t + 1 < nk)
        def _():
            ns = (s + 1) % 3
            if not first:
                wait_out(ns)
            start_in(t + 1, ns)
        wait_in(s)
        compute(s)
        start_out(t, s)

    start_in(0, 0)
    step(0, 0, True)
    step(1, 1, True)
    step(2, 2, False)

    def outer(j, _):
        step(j, 0, False)

        @pl.when(j + 1 < nk)
        def _():
            step(j + 1, 1, False)

        @pl.when(j + 2 < nk)
        def _():
            step(j + 2, 2, False)
        return 0

    lax.fori_loop(1, (nk + 2) // 3, lambda i, c: outer(i * 3, c), 0)

    for s in range(3):
        @pl.when(nk - 3 + s >= 0)
        def _():
            wait_out(s)


@jax.jit
def _run(feat_flat, glob_flat, idx_i32):
    mesh = plsc.VectorSubcoreMesh(core_axis_name="c", subcore_axis_name="s")
    k = pl.kernel(
        _body,
        out_type=jax.ShapeDtypeStruct((N * C,), jnp.float32),
        mesh=mesh,
        scratch_types=[
            pltpu.VMEM((B * C,), jnp.float32),
            pltpu.VMEM((RC,), jnp.float32),
            pltpu.VMEM((RC,), jnp.float32),
            pltpu.VMEM((RC,), jnp.float32),
            pltpu.VMEM((R,), jnp.int32),
            pltpu.VMEM((R,), jnp.int32),
            pltpu.VMEM((R,), jnp.int32),
            pltpu.SemaphoreType.DMA((3,)),
            pltpu.SemaphoreType.DMA((3,)),
            pltpu.SemaphoreType.DMA((3,)),
        ],
    )
    return k(feat_flat, glob_flat, idx_i32)


def kernel(input_features, input_features_global, batch_index):
    feat_flat = input_features.reshape(N * C)
    glob_flat = input_features_global.reshape(B * C)
    idx_i32 = batch_index.astype(jnp.int32)
    out = _run(feat_flat, glob_flat, idx_i32)
    return out.reshape(N, C)
